# baseline (speedup 1.0000x reference)
import functools

import jax
import jax.numpy as jnp
from jax import lax
from jax.experimental import pallas as pl
from jax.experimental.pallas import tpu as pltpu
from jax.experimental.pallas import tpu_sc as plsc

N = 10000
E = 320000
DF = 128
H = 64
TP = 32
B = 16384

NC, NS = 2, 16
EPT = E // (NC * NS)
CHUNK = 80
NCHUNK = EPT // CHUNK
NPAD = 10240
RPT = NPAD // NS
DPT = NPAD // NS
BPT = B // (NC * NS)
GCH = 128
NGCH = BPT // GCH

ECH = 125
ENC = EPT // ECH

_mesh = functools.partial(
    plsc.VectorSubcoreMesh, core_axis_name="c", subcore_axis_name="s")


def _sc_degree(dst_r):
  @functools.partial(
      pl.kernel,
      out_type=jax.ShapeDtypeStruct((NC, NPAD), jnp.float32),
      mesh=_mesh(),
      scratch_types=[
          pltpu.VMEM((ENC, ECH), jnp.int32),
          pltpu.VMEM((DPT,), jnp.float32),
          pltpu.VMEM((128,), jnp.float32),
          pltpu.VMEM_SHARED((NPAD,), jnp.float32),
      ],
  )
  def k(dst_hbm, out_hbm, dv, zb, ones, acc):
    ci = lax.axis_index("c")
    si = lax.axis_index("s")

    def fill_zero(i, c):
      zb[pl.ds(i * 16, 16)] = jnp.zeros((16,), jnp.float32)
      return c
    lax.fori_loop(0, DPT // 16, fill_zero, 0)
    for t in range(128 // 16):
      ones[pl.ds(t * 16, 16)] = jnp.full((16,), 1.0, jnp.float32)

    pltpu.sync_copy(zb, acc.at[pl.ds(si * DPT, DPT)])
    plsc.subcore_barrier()
    pltpu.sync_copy(dst_hbm.at[ci, si], dv)

    def body(j, c):
      pltpu.sync_copy(ones.at[pl.ds(0, ECH)], acc.at[dv.at[j]], add=True)
      return c
    lax.fori_loop(0, ENC, body, 0)
    plsc.subcore_barrier()

    @pl.when(si == 0)
    def _():
      pltpu.sync_copy(acc, out_hbm.at[ci])

  return k(dst_r)


NBUF = 2
ZROWS = 64


def _sc_scatter(g, src_r, dst_r):
  @functools.partial(
      pl.kernel,
      out_type=jax.ShapeDtypeStruct((NC, NPAD, H), jnp.float32),
      mesh=_mesh(),
      compiler_params=pltpu.CompilerParams(use_tc_tiling_on_sc=False),
      scratch_types=[
          pltpu.VMEM((ENC, ECH), jnp.int32),
          pltpu.VMEM((ENC, ECH), jnp.int32),
          pltpu.VMEM((NBUF, ECH, H), jnp.float32),
          pltpu.VMEM((ZROWS, H), jnp.float32),
          pltpu.VMEM_SHARED((NPAD, H), jnp.float32),
          pltpu.VMEM_SHARED((NPAD, H), jnp.float32),
          pltpu.SemaphoreType.DMA,
      ] + [pltpu.SemaphoreType.DMA] * (2 * NBUF),
  )
  def k(g_hbm, src_hbm, dst_hbm, out_hbm, sv, dv, gb, zb, acc, gtab, semt,
        *sems):
    semg = sems[:NBUF]
    sems_ = sems[NBUF:]
    ci = lax.axis_index("c")
    si = lax.axis_index("s")

    pltpu.async_copy(g_hbm.at[pl.ds(si * 625, 625)],
                     gtab.at[pl.ds(si * 625, 625)], semt)

    def fill_zero(i, c):
      for t in range(H // 16):
        zb[i, pl.ds(t * 16, 16)] = jnp.zeros((16,), jnp.float32)
      return c
    lax.fori_loop(0, ZROWS, fill_zero, 0)
    for q in range(RPT // ZROWS):
      pltpu.sync_copy(zb, acc.at[pl.ds(si * RPT + q * ZROWS, ZROWS)])

    pltpu.sync_copy(src_hbm.at[ci, si], sv)
    pltpu.sync_copy(dst_hbm.at[ci, si], dv)
    pltpu.make_async_copy(g_hbm.at[pl.ds(si * 625, 625)],
                          gtab.at[pl.ds(si * 625, 625)], semt).wait()
    plsc.subcore_barrier()

    for t in range(NBUF):
      pltpu.async_copy(gtab.at[sv.at[t]], gb.at[t], semg[t])

    def body(i, c):
      for t in range(NBUF):
        j = NBUF * i + t
        pltpu.make_async_copy(g_hbm.at[sv.at[j]], gb.at[t], semg[t]).wait()
        pltpu.async_copy(gb.at[t], acc.at[dv.at[j]], sems_[t], add=True)
        pltpu.make_async_copy(g_hbm.at[sv.at[j]], gb.at[t], sems_[t]).wait()

        @pl.when(j + NBUF < ENC)
        def _():
          pltpu.async_copy(gtab.at[sv.at[j + NBUF]], gb.at[t], semg[t])
      return c
    lax.fori_loop(0, ENC // NBUF, body, 0)
    plsc.subcore_barrier()

    pltpu.sync_copy(acc.at[pl.ds(si * RPT, RPT)],
                    out_hbm.at[ci, pl.ds(si * RPT, RPT)])

  return k(g, src_r, dst_r)


def _sc_gather2(h3, oi_r, di_r):
  @functools.partial(
      pl.kernel,
      out_type=[jax.ShapeDtypeStruct((B, H), jnp.float32),
                jax.ShapeDtypeStruct((B, H), jnp.float32)],
      mesh=_mesh(),
      compiler_params=pltpu.CompilerParams(use_tc_tiling_on_sc=False),
      scratch_types=[
          pltpu.VMEM((NGCH, GCH), jnp.int32),
          pltpu.VMEM((NGCH, GCH), jnp.int32),
          pltpu.VMEM((GCH, H), jnp.float32),
          pltpu.VMEM((GCH, H), jnp.float32),
          pltpu.SemaphoreType.DMA,
          pltpu.SemaphoreType.DMA,
      ],
  )
  def k(h_hbm, oi_hbm, di_hbm, oout, dout, ob, db, gb0, gb1, sem0, sem1):
    ci = lax.axis_index("c")
    si = lax.axis_index("s")
    base = (ci * NS + si) * BPT
    pltpu.sync_copy(oi_hbm.at[ci, si], ob)
    pltpu.sync_copy(di_hbm.at[ci, si], db)

    bufs = (gb0, gb1)
    sems = (sem0, sem1)
    work = [(ob, oout, j) for j in range(NGCH)] + \
           [(db, dout, j) for j in range(NGCH)]
    pltpu.async_copy(h_hbm.at[work[0][0].at[work[0][2]]], bufs[0], sems[0])
    for t, (idx, out, j) in enumerate(work):
      if t + 1 < len(work):
        nidx, _, nj = work[t + 1]
        pltpu.async_copy(h_hbm.at[nidx.at[nj]], bufs[(t + 1) % 2],
                         sems[(t + 1) % 2])
      pltpu.make_async_copy(h_hbm.at[idx.at[j]], bufs[t % 2],
                            sems[t % 2]).wait()
      pltpu.sync_copy(bufs[t % 2], out.at[pl.ds(base + j * GCH, GCH)])

  return k(h3, oi_r, di_r)


_RB = 2000
_QB = 2048


def _dis(d0, d1):
  return lax.rsqrt(d0 + d1 + 1.0)


def _tc_pre(x, w, d0, d1):
  def body(x_ref, w_ref, d0_ref, d1_ref, o_ref):
    dis = _dis(d0_ref[...], d1_ref[...])
    o_ref[...] = dis * jnp.dot(x_ref[...], w_ref[...],
                               preferred_element_type=jnp.float32)

  return pl.pallas_call(
      body,
      grid=(N // _RB,),
      in_specs=[
          pl.BlockSpec((_RB, DF), lambda i: (i, 0)),
          pl.BlockSpec((DF, H), lambda i: (0, 0)),
          pl.BlockSpec((_RB, 1), lambda i: (i, 0)),
          pl.BlockSpec((_RB, 1), lambda i: (i, 0)),
      ],
      out_specs=pl.BlockSpec((_RB, H), lambda i: (i, 0)),
      out_shape=jax.ShapeDtypeStruct((N, H), jnp.float32),
  )(x, w, d0, d1)


def _tc_mid(s, g, d0, d1, b, w):
  def body(s0_ref, s1_ref, g_ref, d0_ref, d1_ref, b_ref, w_ref, o_ref):
    dis = _dis(d0_ref[...], d1_ref[...])
    h = jnp.maximum(
        dis * (s0_ref[0] + s1_ref[0] + g_ref[...]) + b_ref[...], 0.0)
    o_ref[...] = dis * jnp.dot(h, w_ref[...],
                               preferred_element_type=jnp.float32)

  return pl.pallas_call(
      body,
      grid=(N // _RB,),
      in_specs=[
          pl.BlockSpec((1, _RB, H), lambda i: (0, i, 0)),
          pl.BlockSpec((1, _RB, H), lambda i: (1, i, 0)),
          pl.BlockSpec((_RB, H), lambda i: (i, 0)),
          pl.BlockSpec((_RB, 1), lambda i: (i, 0)),
          pl.BlockSpec((_RB, 1), lambda i: (i, 0)),
          pl.BlockSpec((1, H), lambda i: (0, 0)),
          pl.BlockSpec((H, H), lambda i: (0, 0)),
      ],
      out_specs=pl.BlockSpec((_RB, H), lambda i: (i, 0)),
      out_shape=jax.ShapeDtypeStruct((N, H), jnp.float32),
  )(s, s, g, d0, d1, b, w)


def _tc_post(s, g, d0, d1, b):
  def body(s0_ref, s1_ref, g_ref, d0_ref, d1_ref, b_ref, o_ref):
    dis = _dis(d0_ref[...], d1_ref[...])
    o_ref[...] = jnp.maximum(
        dis * (s0_ref[0] + s1_ref[0] + g_ref[...]) + b_ref[...], 0.0)

  return pl.pallas_call(
      body,
      grid=(N // _RB,),
      in_specs=[
          pl.BlockSpec((1, _RB, H), lambda i: (0, i, 0)),
          pl.BlockSpec((1, _RB, H), lambda i: (1, i, 0)),
          pl.BlockSpec((_RB, H), lambda i: (i, 0)),
          pl.BlockSpec((_RB, 1), lambda i: (i, 0)),
          pl.BlockSpec((_RB, 1), lambda i: (i, 0)),
          pl.BlockSpec((1, H), lambda i: (0, 0)),
      ],
      out_specs=pl.BlockSpec((_RB, H), lambda i: (i, 0)),
      out_shape=jax.ShapeDtypeStruct((N, H), jnp.float32),
  )(s, s, g, d0, d1, b)


def _embed(ids, table_ref, nrows):
  acc = jnp.zeros((ids.shape[0], TP), jnp.float32)
  for v in range(nrows):
    row = table_ref[v:v + 1, :]
    acc = acc + jnp.where(ids == v, 1.0, 0.0) * row
  return acc


def _tc_head(orig, dest, day, time, mode, day_t, time_t, mode_t,
             wta, wtb, bt, wp1a, wp1b, wp1c, wp1d, bp1,
             wp2, bp2, wp3, bp3, wp4, bp4):
  def body(orig_ref, dest_ref, day_ref, time_ref, mode_ref,
           dayt_ref, timet_ref, modet_ref, wta_ref, wtb_ref, bt_ref,
           wp1a_ref, wp1b_ref, wp1c_ref, wp1d_ref, bp1_ref,
           wp2_ref, bp2_ref, wp3_ref, bp3_ref, wp4_ref, bp4_ref, o_ref):
    day_e = _embed(day_ref[...], dayt_ref, 2)
    time_e = _embed(time_ref[...], timet_ref, 5)
    mode_e = _embed(mode_ref[...], modet_ref, 3)
    dot = functools.partial(jnp.dot, preferred_element_type=jnp.float32)
    t = jnp.maximum(dot(day_e, wta_ref[...]) + dot(time_e, wtb_ref[...])
                    + bt_ref[...], 0.0)
    z = (dot(orig_ref[...], wp1a_ref[...]) + dot(dest_ref[...], wp1b_ref[...])
         + dot(t, wp1c_ref[...]) + dot(mode_e, wp1d_ref[...]) + bp1_ref[...])
    z = jnp.maximum(z, 0.0)
    z = jnp.maximum(dot(z, wp2_ref[...]) + bp2_ref[...], 0.0)
    z = jnp.maximum(dot(z, wp3_ref[...]) + bp3_ref[...], 0.0)
    o_ref[...] = jax.nn.sigmoid(dot(z, wp4_ref[...]) + bp4_ref[...])

  full = lambda shape: pl.BlockSpec(shape, lambda i: tuple(0 for _ in shape))
  blk = lambda cols: pl.BlockSpec((_QB, cols), lambda i: (i, 0))
  return pl.pallas_call(
      body,
      grid=(B // _QB,),
      in_specs=[
          blk(H), blk(H), blk(1), blk(1), blk(1),
          full((8, TP)), full((8, TP)), full((8, TP)),
          full((TP, H)), full((TP, H)), full((1, H)),
          full((H, 2 * H)), full((H, 2 * H)), full((H, 2 * H)),
          full((TP, 2 * H)), full((1, 2 * H)),
          full((2 * H, H)), full((1, H)),
          full((H, H // 2)), full((1, H // 2)),
          full((H // 2, 1)), full((1, 1)),
      ],
      out_specs=blk(1),
      out_shape=jax.ShapeDtypeStruct((B, 1), jnp.float32),
  )(orig, dest, day, time, mode, day_t, time_t, mode_t,
    wta, wtb, bt, wp1a, wp1b, wp1c, wp1d, bp1,
    wp2, bp2, wp3, bp3, wp4, bp4)


def _pad8(t):
  out = jnp.zeros((8, t.shape[1]), t.dtype)
  return out.at[:t.shape[0]].set(t)


def kernel(x, edge_index, origin_ids, destination_ids, day_type_ids,
           time_period_ids, mode_ids, W1, b1, W2, b2, W3, b3,
           day_emb, time_emb, mode_emb, Wt, bt,
           Wp1, bp1, Wp2, bp2, Wp3, bp3, Wp4, bp4):
  src_r = edge_index[0].reshape(NC, NS, ENC, ECH)
  dst_r = edge_index[1].reshape(NC, NS, ENC, ECH)

  degs = _sc_degree(dst_r)
  d0 = degs[0, :N].reshape(N, 1)
  d1 = degs[1, :N].reshape(N, 1)

  g1 = _tc_pre(x, W1, d0, d1)
  s = _sc_scatter(g1, src_r, dst_r)
  g2 = _tc_mid(s, g1, d0, d1, b1.reshape(1, H), W2)
  s = _sc_scatter(g2, src_r, dst_r)
  g3 = _tc_mid(s, g2, d0, d1, b2.reshape(1, H), W3)
  s = _sc_scatter(g3, src_r, dst_r)
  h3 = _tc_post(s, g3, d0, d1, b3.reshape(1, H))

  oi_r = origin_ids.reshape(NC, NS, NGCH, GCH)
  di_r = destination_ids.reshape(NC, NS, NGCH, GCH)
  orig, dest = _sc_gather2(h3, oi_r, di_r)

  score = _tc_head(
      orig, dest,
      day_type_ids.reshape(B, 1), time_period_ids.reshape(B, 1),
      mode_ids.reshape(B, 1),
      _pad8(day_emb), _pad8(time_emb), _pad8(mode_emb),
      Wt[:TP], Wt[TP:], bt.reshape(1, H),
      Wp1[:H], Wp1[H:2 * H], Wp1[2 * H:3 * H], Wp1[3 * H:], bp1.reshape(1, 2 * H),
      Wp2, bp2.reshape(1, H), Wp3, bp3.reshape(1, H // 2),
      Wp4, bp4.reshape(1, 1))
  return score


# speedup vs baseline: 1.2044x; 1.2044x over previous
import functools

import jax
import jax.numpy as jnp
from jax import lax
from jax.experimental import pallas as pl
from jax.experimental.pallas import tpu as pltpu
from jax.experimental.pallas import tpu_sc as plsc

N = 10000
E = 320000
DF = 128
H = 64
TP = 32
B = 16384

NC, NS = 2, 16
EPT = E // (NC * NS)
CHUNK = 80
NCHUNK = EPT // CHUNK
NPAD = 10240
RPT = NPAD // NS
DPT = NPAD // NS
BPT = B // (NC * NS)
GCH = 128
NGCH = BPT // GCH

ECH = 125
ENC = EPT // ECH

_mesh = functools.partial(
    plsc.VectorSubcoreMesh, core_axis_name="c", subcore_axis_name="s")


def _sc_degree(dst_r):
  @functools.partial(
      pl.kernel,
      out_type=jax.ShapeDtypeStruct((NC, NPAD), jnp.float32),
      mesh=_mesh(),
      scratch_types=[
          pltpu.VMEM((ENC, ECH), jnp.int32),
          pltpu.VMEM((DPT,), jnp.float32),
          pltpu.VMEM((128,), jnp.float32),
          pltpu.VMEM_SHARED((NPAD,), jnp.float32),
      ],
  )
  def k(dst_hbm, out_hbm, dv, zb, ones, acc):
    ci = lax.axis_index("c")
    si = lax.axis_index("s")

    def fill_zero(i, c):
      zb[pl.ds(i * 16, 16)] = jnp.zeros((16,), jnp.float32)
      return c
    lax.fori_loop(0, DPT // 16, fill_zero, 0)
    for t in range(128 // 16):
      ones[pl.ds(t * 16, 16)] = jnp.full((16,), 1.0, jnp.float32)

    pltpu.sync_copy(zb, acc.at[pl.ds(si * DPT, DPT)])
    plsc.subcore_barrier()
    pltpu.sync_copy(dst_hbm.at[ci, si], dv)

    def body(j, c):
      pltpu.sync_copy(ones.at[pl.ds(0, ECH)], acc.at[dv.at[j]], add=True)
      return c
    lax.fori_loop(0, ENC, body, 0)
    plsc.subcore_barrier()

    @pl.when(si == 0)
    def _():
      pltpu.sync_copy(acc, out_hbm.at[ci])

  return k(dst_r)


NBUF = 5
ZROWS = 128


def _sc_scatter(g, src_r, dst_r):
  @functools.partial(
      pl.kernel,
      out_type=jax.ShapeDtypeStruct((NC, NPAD, H), jnp.float32),
      mesh=_mesh(),
      compiler_params=pltpu.CompilerParams(use_tc_tiling_on_sc=False),
      scratch_types=[
          pltpu.VMEM((ENC, ECH), jnp.int32),
          pltpu.VMEM((ENC, ECH), jnp.int32),
          pltpu.VMEM((NBUF, ECH, H), jnp.float32),
          pltpu.VMEM((ZROWS, H), jnp.float32),
          pltpu.VMEM_SHARED((NPAD, H), jnp.float32),
      ] + [pltpu.SemaphoreType.DMA] * (2 * NBUF),
  )
  def k(g_hbm, src_hbm, dst_hbm, out_hbm, sv, dv, gb, zb, acc, *sems):
    semg = sems[:NBUF]
    sems_ = sems[NBUF:]
    ci = lax.axis_index("c")
    si = lax.axis_index("s")

    def fill_zero(i, c):
      for t in range(H // 16):
        zb[i, pl.ds(t * 16, 16)] = jnp.zeros((16,), jnp.float32)
      return c
    lax.fori_loop(0, ZROWS, fill_zero, 0)
    for q in range(RPT // ZROWS):
      pltpu.sync_copy(zb, acc.at[pl.ds(si * RPT + q * ZROWS, ZROWS)])
    plsc.subcore_barrier()

    pltpu.sync_copy(src_hbm.at[ci, si], sv)
    pltpu.sync_copy(dst_hbm.at[ci, si], dv)

    for t in range(NBUF):
      pltpu.async_copy(g_hbm.at[sv.at[t]], gb.at[t], semg[t])

    def body(i, c):
      for t in range(NBUF):
        j = NBUF * i + t
        pltpu.make_async_copy(g_hbm.at[sv.at[j]], gb.at[t], semg[t]).wait()
        pltpu.async_copy(gb.at[t], acc.at[dv.at[j]], sems_[t], add=True)
        pltpu.make_async_copy(g_hbm.at[sv.at[j]], gb.at[t], sems_[t]).wait()

        @pl.when(j + NBUF < ENC)
        def _():
          pltpu.async_copy(g_hbm.at[sv.at[j + NBUF]], gb.at[t], semg[t])
      return c
    lax.fori_loop(0, ENC // NBUF, body, 0)
    plsc.subcore_barrier()

    pltpu.sync_copy(acc.at[pl.ds(si * RPT, RPT)],
                    out_hbm.at[ci, pl.ds(si * RPT, RPT)])

  return k(g, src_r, dst_r)


def _sc_gather2(h3, oi_r, di_r):
  @functools.partial(
      pl.kernel,
      out_type=[jax.ShapeDtypeStruct((B, H), jnp.float32),
                jax.ShapeDtypeStruct((B, H), jnp.float32)],
      mesh=_mesh(),
      compiler_params=pltpu.CompilerParams(use_tc_tiling_on_sc=False),
      scratch_types=[
          pltpu.VMEM((NGCH, GCH), jnp.int32),
          pltpu.VMEM((NGCH, GCH), jnp.int32),
          pltpu.VMEM((GCH, H), jnp.float32),
          pltpu.VMEM((GCH, H), jnp.float32),
          pltpu.SemaphoreType.DMA,
          pltpu.SemaphoreType.DMA,
      ],
  )
  def k(h_hbm, oi_hbm, di_hbm, oout, dout, ob, db, gb0, gb1, sem0, sem1):
    ci = lax.axis_index("c")
    si = lax.axis_index("s")
    base = (ci * NS + si) * BPT
    pltpu.sync_copy(oi_hbm.at[ci, si], ob)
    pltpu.sync_copy(di_hbm.at[ci, si], db)

    bufs = (gb0, gb1)
    sems = (sem0, sem1)
    work = [(ob, oout, j) for j in range(NGCH)] + \
           [(db, dout, j) for j in range(NGCH)]
    pltpu.async_copy(h_hbm.at[work[0][0].at[work[0][2]]], bufs[0], sems[0])
    for t, (idx, out, j) in enumerate(work):
      if t + 1 < len(work):
        nidx, _, nj = work[t + 1]
        pltpu.async_copy(h_hbm.at[nidx.at[nj]], bufs[(t + 1) % 2],
                         sems[(t + 1) % 2])
      pltpu.make_async_copy(h_hbm.at[idx.at[j]], bufs[t % 2],
                            sems[t % 2]).wait()
      pltpu.sync_copy(bufs[t % 2], out.at[pl.ds(base + j * GCH, GCH)])

  return k(h3, oi_r, di_r)


_RB = 2000
_QB = 2048


def _dis(d0, d1):
  return lax.rsqrt(d0 + d1 + 1.0)


def _tc_pre(x, w, d0, d1):
  def body(x_ref, w_ref, d0_ref, d1_ref, o_ref):
    dis = _dis(d0_ref[...], d1_ref[...])
    o_ref[...] = dis * jnp.dot(x_ref[...], w_ref[...],
                               preferred_element_type=jnp.float32)

  return pl.pallas_call(
      body,
      grid=(N // _RB,),
      in_specs=[
          pl.BlockSpec((_RB, DF), lambda i: (i, 0)),
          pl.BlockSpec((DF, H), lambda i: (0, 0)),
          pl.BlockSpec((_RB, 1), lambda i: (i, 0)),
          pl.BlockSpec((_RB, 1), lambda i: (i, 0)),
      ],
      out_specs=pl.BlockSpec((_RB, H), lambda i: (i, 0)),
      out_shape=jax.ShapeDtypeStruct((N, H), jnp.float32),
  )(x, w, d0, d1)


def _tc_mid(s, g, d0, d1, b, w):
  def body(s0_ref, s1_ref, g_ref, d0_ref, d1_ref, b_ref, w_ref, o_ref):
    dis = _dis(d0_ref[...], d1_ref[...])
    h = jnp.maximum(
        dis * (s0_ref[0] + s1_ref[0] + g_ref[...]) + b_ref[...], 0.0)
    o_ref[...] = dis * jnp.dot(h, w_ref[...],
                               preferred_element_type=jnp.float32)

  return pl.pallas_call(
      body,
      grid=(N // _RB,),
      in_specs=[
          pl.BlockSpec((1, _RB, H), lambda i: (0, i, 0)),
          pl.BlockSpec((1, _RB, H), lambda i: (1, i, 0)),
          pl.BlockSpec((_RB, H), lambda i: (i, 0)),
          pl.BlockSpec((_RB, 1), lambda i: (i, 0)),
          pl.BlockSpec((_RB, 1), lambda i: (i, 0)),
          pl.BlockSpec((1, H), lambda i: (0, 0)),
          pl.BlockSpec((H, H), lambda i: (0, 0)),
      ],
      out_specs=pl.BlockSpec((_RB, H), lambda i: (i, 0)),
      out_shape=jax.ShapeDtypeStruct((N, H), jnp.float32),
  )(s, s, g, d0, d1, b, w)


def _tc_post(s, g, d0, d1, b):
  def body(s0_ref, s1_ref, g_ref, d0_ref, d1_ref, b_ref, o_ref):
    dis = _dis(d0_ref[...], d1_ref[...])
    o_ref[...] = jnp.maximum(
        dis * (s0_ref[0] + s1_ref[0] + g_ref[...]) + b_ref[...], 0.0)

  return pl.pallas_call(
      body,
      grid=(N // _RB,),
      in_specs=[
          pl.BlockSpec((1, _RB, H), lambda i: (0, i, 0)),
          pl.BlockSpec((1, _RB, H), lambda i: (1, i, 0)),
          pl.BlockSpec((_RB, H), lambda i: (i, 0)),
          pl.BlockSpec((_RB, 1), lambda i: (i, 0)),
          pl.BlockSpec((_RB, 1), lambda i: (i, 0)),
          pl.BlockSpec((1, H), lambda i: (0, 0)),
      ],
      out_specs=pl.BlockSpec((_RB, H), lambda i: (i, 0)),
      out_shape=jax.ShapeDtypeStruct((N, H), jnp.float32),
  )(s, s, g, d0, d1, b)


def _embed(ids, table_ref, nrows):
  acc = jnp.zeros((ids.shape[0], TP), jnp.float32)
  for v in range(nrows):
    row = table_ref[v:v + 1, :]
    acc = acc + jnp.where(ids == v, 1.0, 0.0) * row
  return acc


def _tc_head(orig, dest, day, time, mode, day_t, time_t, mode_t,
             wta, wtb, bt, wp1a, wp1b, wp1c, wp1d, bp1,
             wp2, bp2, wp3, bp3, wp4, bp4):
  def body(orig_ref, dest_ref, day_ref, time_ref, mode_ref,
           dayt_ref, timet_ref, modet_ref, wta_ref, wtb_ref, bt_ref,
           wp1a_ref, wp1b_ref, wp1c_ref, wp1d_ref, bp1_ref,
           wp2_ref, bp2_ref, wp3_ref, bp3_ref, wp4_ref, bp4_ref, o_ref):
    day_e = _embed(day_ref[...], dayt_ref, 2)
    time_e = _embed(time_ref[...], timet_ref, 5)
    mode_e = _embed(mode_ref[...], modet_ref, 3)
    dot = functools.partial(jnp.dot, preferred_element_type=jnp.float32)
    t = jnp.maximum(dot(day_e, wta_ref[...]) + dot(time_e, wtb_ref[...])
                    + bt_ref[...], 0.0)
    z = (dot(orig_ref[...], wp1a_ref[...]) + dot(dest_ref[...], wp1b_ref[...])
         + dot(t, wp1c_ref[...]) + dot(mode_e, wp1d_ref[...]) + bp1_ref[...])
    z = jnp.maximum(z, 0.0)
    z = jnp.maximum(dot(z, wp2_ref[...]) + bp2_ref[...], 0.0)
    z = jnp.maximum(dot(z, wp3_ref[...]) + bp3_ref[...], 0.0)
    o_ref[...] = jax.nn.sigmoid(dot(z, wp4_ref[...]) + bp4_ref[...])

  full = lambda shape: pl.BlockSpec(shape, lambda i: tuple(0 for _ in shape))
  blk = lambda cols: pl.BlockSpec((_QB, cols), lambda i: (i, 0))
  return pl.pallas_call(
      body,
      grid=(B // _QB,),
      in_specs=[
          blk(H), blk(H), blk(1), blk(1), blk(1),
          full((8, TP)), full((8, TP)), full((8, TP)),
          full((TP, H)), full((TP, H)), full((1, H)),
          full((H, 2 * H)), full((H, 2 * H)), full((H, 2 * H)),
          full((TP, 2 * H)), full((1, 2 * H)),
          full((2 * H, H)), full((1, H)),
          full((H, H // 2)), full((1, H // 2)),
          full((H // 2, 1)), full((1, 1)),
      ],
      out_specs=blk(1),
      out_shape=jax.ShapeDtypeStruct((B, 1), jnp.float32),
  )(orig, dest, day, time, mode, day_t, time_t, mode_t,
    wta, wtb, bt, wp1a, wp1b, wp1c, wp1d, bp1,
    wp2, bp2, wp3, bp3, wp4, bp4)


def _pad8(t):
  out = jnp.zeros((8, t.shape[1]), t.dtype)
  return out.at[:t.shape[0]].set(t)


def kernel(x, edge_index, origin_ids, destination_ids, day_type_ids,
           time_period_ids, mode_ids, W1, b1, W2, b2, W3, b3,
           day_emb, time_emb, mode_emb, Wt, bt,
           Wp1, bp1, Wp2, bp2, Wp3, bp3, Wp4, bp4):
  src_r = edge_index[0].reshape(NC, NS, ENC, ECH)
  dst_r = edge_index[1].reshape(NC, NS, ENC, ECH)

  degs = _sc_degree(dst_r)
  d0 = degs[0, :N].reshape(N, 1)
  d1 = degs[1, :N].reshape(N, 1)

  g1 = _tc_pre(x, W1, d0, d1)
  s = _sc_scatter(g1, src_r, dst_r)
  g2 = _tc_mid(s, g1, d0, d1, b1.reshape(1, H), W2)
  s = _sc_scatter(g2, src_r, dst_r)
  g3 = _tc_mid(s, g2, d0, d1, b2.reshape(1, H), W3)
  s = _sc_scatter(g3, src_r, dst_r)
  h3 = _tc_post(s, g3, d0, d1, b3.reshape(1, H))

  oi_r = origin_ids.reshape(NC, NS, NGCH, GCH)
  di_r = destination_ids.reshape(NC, NS, NGCH, GCH)
  orig, dest = _sc_gather2(h3, oi_r, di_r)

  score = _tc_head(
      orig, dest,
      day_type_ids.reshape(B, 1), time_period_ids.reshape(B, 1),
      mode_ids.reshape(B, 1),
      _pad8(day_emb), _pad8(time_emb), _pad8(mode_emb),
      Wt[:TP], Wt[TP:], bt.reshape(1, H),
      Wp1[:H], Wp1[H:2 * H], Wp1[2 * H:3 * H], Wp1[3 * H:], bp1.reshape(1, 2 * H),
      Wp2, bp2.reshape(1, H), Wp3, bp3.reshape(1, H // 2),
      Wp4, bp4.reshape(1, 1))
  return score
